# Initial kernel scaffold; baseline (speedup 1.0000x reference)
import jax
import jax.numpy as jnp
from jax.experimental import pallas as pl


def kernel(x, edge_index, batch, W1, att_src1, att_dst1, bias1, W2, att_src2, att_dst2, bias2):
    raise NotImplementedError("write your pallas kernel here")


# speedup vs baseline: 2.9360x; 2.9360x over previous
import jax
import jax.numpy as jnp
from jax.experimental import pallas as pl

_N = 10000
_NP = 10240
_BLK = 1024
_G = 128


def _dense_att_body(x_ref, w_ref, asrc_ref, adst_ref, h_ref, s_ref, d_ref):
    h = jnp.dot(x_ref[...], w_ref[...], preferred_element_type=jnp.float32)
    h_ref[...] = h
    s_ref[...] = jnp.dot(h, asrc_ref[...], preferred_element_type=jnp.float32)
    d_ref[...] = jnp.dot(h, adst_ref[...], preferred_element_type=jnp.float32)


def _elu_dense_att_body(a_ref, b_ref, w_ref, asrc_ref, adst_ref, h_ref, s_ref, d_ref):
    t = a_ref[...] + b_ref[...]
    t = jnp.where(t > 0, t, jnp.exp(jnp.minimum(t, 0.0)) - 1.0)
    h = jnp.dot(t, w_ref[...], preferred_element_type=jnp.float32)
    h_ref[...] = h
    s_ref[...] = jnp.dot(h, asrc_ref[...], preferred_element_type=jnp.float32)
    d_ref[...] = jnp.dot(h, adst_ref[...], preferred_element_type=jnp.float32)


def _dense_att(x, W, Asrc, Adst):
    n, k = x.shape
    m = W.shape[1]
    hd = Asrc.shape[1]
    return pl.pallas_call(
        _dense_att_body,
        grid=(n // _BLK,),
        in_specs=[
            pl.BlockSpec((_BLK, k), lambda i: (i, 0)),
            pl.BlockSpec((k, m), lambda i: (0, 0)),
            pl.BlockSpec((m, hd), lambda i: (0, 0)),
            pl.BlockSpec((m, hd), lambda i: (0, 0)),
        ],
        out_specs=[
            pl.BlockSpec((_BLK, m), lambda i: (i, 0)),
            pl.BlockSpec((_BLK, hd), lambda i: (i, 0)),
            pl.BlockSpec((_BLK, hd), lambda i: (i, 0)),
        ],
        out_shape=[
            jax.ShapeDtypeStruct((n, m), jnp.float32),
            jax.ShapeDtypeStruct((n, hd), jnp.float32),
            jax.ShapeDtypeStruct((n, hd), jnp.float32),
        ],
    )(x, W, Asrc, Adst)


def _elu_dense_att(a, b, W, Asrc, Adst):
    n, k = a.shape
    m = W.shape[1]
    hd = Asrc.shape[1]
    return pl.pallas_call(
        _elu_dense_att_body,
        grid=(n // _BLK,),
        in_specs=[
            pl.BlockSpec((_BLK, k), lambda i: (i, 0)),
            pl.BlockSpec((1, k), lambda i: (0, 0)),
            pl.BlockSpec((k, m), lambda i: (0, 0)),
            pl.BlockSpec((m, hd), lambda i: (0, 0)),
            pl.BlockSpec((m, hd), lambda i: (0, 0)),
        ],
        out_specs=[
            pl.BlockSpec((_BLK, m), lambda i: (i, 0)),
            pl.BlockSpec((_BLK, hd), lambda i: (i, 0)),
            pl.BlockSpec((_BLK, hd), lambda i: (i, 0)),
        ],
        out_shape=[
            jax.ShapeDtypeStruct((n, m), jnp.float32),
            jax.ShapeDtypeStruct((n, hd), jnp.float32),
            jax.ShapeDtypeStruct((n, hd), jnp.float32),
        ],
    )(a, b, W, Asrc, Adst)


def _expand_att(att, heads, out_ch):
    a = att.reshape(heads, out_ch)
    eye = jnp.eye(heads, dtype=a.dtype)
    return (a[:, :, None] * eye[:, None, :]).reshape(heads * out_ch, heads)


def _edge_pass(h, a_src, a_dst, src, dst, heads, out_ch):
    alpha = a_src[src] + a_dst[dst]
    alpha = jnp.where(alpha > 0, alpha, 0.2 * alpha)
    amax = jax.ops.segment_max(alpha, dst, num_segments=_N)
    alpha = jnp.exp(alpha - amax[dst])
    denom = jax.ops.segment_sum(alpha, dst, num_segments=_N)
    alpha = alpha / (denom[dst] + 1e-16)
    msg = h[src].reshape(-1, heads, out_ch) * alpha[:, :, None]
    return jax.ops.segment_sum(msg.reshape(-1, heads * out_ch), dst,
                               num_segments=_N)


def kernel(x, edge_index, batch, W1, att_src1, att_dst1, bias1,
           W2, att_src2, att_dst2, bias2):
    loop = jnp.arange(_N, dtype=edge_index.dtype)
    ei = jnp.concatenate([edge_index, jnp.stack([loop, loop])], axis=1)
    src, dst = ei[0], ei[1]

    xp = jnp.pad(x, ((0, _NP - _N), (0, 0)))
    h1, as1, ad1 = _dense_att(xp, W1,
                              _expand_att(att_src1, 8, 8),
                              _expand_att(att_dst1, 8, 8))
    agg1 = _edge_pass(h1[:_N], as1[:_N], ad1[:_N], src, dst, 8, 8)

    agg1p = jnp.pad(agg1, ((0, _NP - _N), (0, 0)))
    h2, as2, ad2 = _elu_dense_att(agg1p, bias1.reshape(1, -1), W2,
                                  _expand_att(att_src2, 1, 2),
                                  _expand_att(att_dst2, 1, 2))
    agg2 = _edge_pass(h2[:_N], as2[:_N], ad2[:_N], src, dst, 1, 2)

    out_nodes = agg2 + bias2
    return jax.ops.segment_sum(out_nodes, batch, num_segments=_G)
